# Initial kernel scaffold; baseline (speedup 1.0000x reference)
import jax
import jax.numpy as jnp
from jax.experimental import pallas as pl


def kernel(current_triples, corrupted_triples, entity_emb, relation_emb):
    raise NotImplementedError("write your pallas kernel here")


# speedup vs baseline: 1.0372x; 1.0372x over previous
import functools

import jax
import jax.numpy as jnp
from jax import lax
from jax.experimental import pallas as pl
from jax.experimental.pallas import tpu as pltpu
from jax.experimental.pallas import tpu_sc as plsc

_DIM = 128
_NTRI = 4096
_MARGIN = 1.0
_C = 0.25

_NC = 2
_NS = 16
_NW = _NC * _NS
_TPW = _NTRI // _NW
_NG = _TPW // 16


def _sc_loss_partials(idx_all, entity_emb, relation_emb):
    mesh = plsc.VectorSubcoreMesh(core_axis_name="c", subcore_axis_name="s")

    @functools.partial(
        pl.kernel,
        out_type=jax.ShapeDtypeStruct((_NW, 48), jnp.float32),
        mesh=mesh,
        compiler_params=pltpu.CompilerParams(needs_layout_passes=False),
        scratch_types=[
            pltpu.VMEM((6, _TPW), jnp.int32),
            pltpu.VMEM((_TPW, _DIM), jnp.float32),
            pltpu.VMEM((_TPW, _DIM), jnp.float32),
            pltpu.VMEM((_TPW, _DIM), jnp.float32),
            pltpu.VMEM((_TPW, _DIM), jnp.float32),
            pltpu.VMEM((_TPW, _DIM), jnp.float32),
            pltpu.VMEM((_TPW, _DIM), jnp.float32),
            pltpu.VMEM((48,), jnp.float32),
            pltpu.SemaphoreType.DMA,
        ],
    )
    def k(idx_hbm, ent_hbm, rel_hbm, out_hbm,
          idx_v, hp_v, rp_v, tp_v, hn_v, rn_v, tn_v, out_v, sem):
        wid = lax.axis_index("s") * _NC + lax.axis_index("c")
        base = wid * _TPW

        pltpu.sync_copy(idx_hbm.at[:, pl.ds(base, _TPW)], idx_v)
        cps = [
            pltpu.async_copy(ent_hbm.at[idx_v.at[0]], hp_v, sem),
            pltpu.async_copy(rel_hbm.at[idx_v.at[1]], rp_v, sem),
            pltpu.async_copy(ent_hbm.at[idx_v.at[2]], tp_v, sem),
            pltpu.async_copy(ent_hbm.at[idx_v.at[3]], hn_v, sem),
            pltpu.async_copy(rel_hbm.at[idx_v.at[4]], rn_v, sem),
            pltpu.async_copy(ent_hbm.at[idx_v.at[5]], tn_v, sem),
        ]
        for cp in cps:
            cp.wait()

        lanes = lax.iota(jnp.int32, 16)
        zero = jnp.zeros((16,), jnp.float32)
        one = jnp.full((16,), 1.0, jnp.float32)

        def sqrt16(x):
            i = lax.bitcast_convert_type(x, jnp.int32)
            i = 0x5F3759DF - lax.shift_right_arithmetic(i, 1)
            y = lax.bitcast_convert_type(i, jnp.float32)
            for _ in range(3):
                y = y * (1.5 - 0.5 * x * y * y)
            return x * y

        def group_body(g, carry):
            acc_m, acc_pe, acc_pr = carry
            rows = g * 16 + lanes

            def dim_body(j, c):
                dd_p, dd_n, hh_p, tt_p, rr_p, hh_n, tt_n, rr_n = c
                cols = jnp.full((16,), j, jnp.int32)
                hp = plsc.load_gather(hp_v, [rows, cols])
                rp = plsc.load_gather(rp_v, [rows, cols])
                tp = plsc.load_gather(tp_v, [rows, cols])
                hn = plsc.load_gather(hn_v, [rows, cols])
                rn = plsc.load_gather(rn_v, [rows, cols])
                tn = plsc.load_gather(tn_v, [rows, cols])
                dp = hp + rp - tp
                dn = hn + rn - tn
                return (dd_p + dp * dp, dd_n + dn * dn,
                        hh_p + hp * hp, tt_p + tp * tp, rr_p + rp * rp,
                        hh_n + hn * hn, tt_n + tn * tn, rr_n + rn * rn)

            dd_p, dd_n, hh_p, tt_p, rr_p, hh_n, tt_n, rr_n = lax.fori_loop(
                0, _DIM, dim_body, (zero, zero, zero, zero, zero, zero, zero, zero))

            pos = sqrt16(dd_p)
            neg = sqrt16(dd_n)
            acc_m = acc_m + jnp.maximum(pos - neg + _MARGIN, 0.0)
            acc_pe = (acc_pe
                      + jnp.maximum(hh_p - one, 0.0) + jnp.maximum(tt_p - one, 0.0)
                      + jnp.maximum(hh_n - one, 0.0) + jnp.maximum(tt_n - one, 0.0))
            acc_pr = acc_pr + jnp.maximum(rr_p - one, 0.0) + jnp.maximum(rr_n - one, 0.0)
            return acc_m, acc_pe, acc_pr

        acc_m, acc_pe, acc_pr = lax.fori_loop(
            0, _NG, group_body, (zero, zero, zero))

        out_v[pl.ds(0, 16)] = acc_m
        out_v[pl.ds(16, 16)] = acc_pe
        out_v[pl.ds(32, 16)] = acc_pr
        pltpu.sync_copy(out_v, out_hbm.at[wid])

    return k(idx_all, entity_emb, relation_emb)


def kernel(current_triples, corrupted_triples, entity_emb, relation_emb):
    idx_all = jnp.concatenate(
        [current_triples.T, corrupted_triples.T], axis=0).astype(jnp.int32)
    parts = _sc_loss_partials(idx_all, entity_emb, relation_emb)
    margin_sum = jnp.sum(parts[:, 0:16])
    ent_pen = jnp.sum(parts[:, 16:32])
    rel_pen = jnp.sum(parts[:, 32:48])
    loss = margin_sum / _NTRI
    return loss + _C * (ent_pen / (4 * _NTRI) + rel_pen / (2 * _NTRI))
